# baseline (speedup 1.0000x reference)
import functools

import jax
import jax.numpy as jnp
from jax import lax
from jax.experimental import pallas as pl
from jax.experimental.pallas import tpu as pltpu
from jax.experimental.pallas import tpu_sc as plsc

N_NODES = 10000
N_EDGES = 320000
D = 128

_info = plsc.get_sparse_core_info()
NC = _info.num_cores
NS = _info.num_subcores
L = _info.num_lanes
NW = NC * NS

CHUNK = 112
K = 3
NCHUNK = 90
EPW = NCHUNK * CHUNK
EPAD = EPW * NW
NG = NCHUNK // K

ACC_ROWS = 10112
ZROWS = ACC_ROWS // NS


def _elu_body(x_ref, w_ref, o_ref):
    j = pl.program_id(1)
    y = x_ref[...] * w_ref[...]
    y = jnp.where(y > 0, y, jnp.exp(y) - 1.0)
    o_ref[...] = y * (1.0 + j.astype(jnp.float32))


def _build_table(graph_embedding, W):
    blk = 1000
    grid = (N_NODES // blk, 2)
    return pl.pallas_call(
        _elu_body,
        grid=grid,
        in_specs=[
            pl.BlockSpec((blk, D), lambda i, j: (i, 0)),
            pl.BlockSpec((1, D), lambda i, j: (0, 0)),
        ],
        out_specs=pl.BlockSpec((blk, D), lambda i, j: (j * (N_NODES // blk) + i, 0)),
        out_shape=jax.ShapeDtypeStruct((2 * N_NODES, D), jnp.float32),
    )(graph_embedding, W)


_mesh = plsc.VectorSubcoreMesh(core_axis_name="c", subcore_axis_name="s")


@functools.partial(
    pl.kernel,
    mesh=_mesh,
    out_type=jax.ShapeDtypeStruct((NC, ACC_ROWS, D), jnp.float32),
    scratch_types=[
        pltpu.VMEM((2, K, 3, CHUNK), jnp.int32),
        pltpu.VMEM((K, CHUNK), jnp.int32),
        pltpu.VMEM((K, CHUNK, D), jnp.float32),
        pltpu.VMEM_SHARED((ACC_ROWS, D), jnp.float32),
        pltpu.SemaphoreType.DMA,
        pltpu.SemaphoreType.DMA,
        pltpu.SemaphoreType.DMA,
        pltpu.SemaphoreType.DMA,
        pltpu.SemaphoreType.DMA,
        pltpu.SemaphoreType.DMA,
        pltpu.SemaphoreType.DMA,
    ],
)
def _sc_gather_scatter(tbl_hbm, sde_hbm, zeros_hbm, out_hbm,
                       sde_v, gidx_v, rows_v, acc_sh,
                       ga, gb, gc, sa, sb, sc, st):
    c = lax.axis_index("c")
    s = lax.axis_index("s")
    w = c * NS + s
    gsems = (ga, gb, gc)
    ssems = (sa, sb, sc)

    pltpu.sync_copy(zeros_hbm.at[pl.ds(s * ZROWS, ZROWS)],
                    acc_sh.at[pl.ds(s * ZROWS, ZROWS)])
    plsc.subcore_barrier()

    def compute_gidx(p, b):
        for k in range(CHUNK // L):
            sl = pl.ds(k * L, L)
            s16 = sde_v[p, b, 0, sl]
            e16 = sde_v[p, b, 2, sl]
            gidx_v[b, sl] = s16 + (1 - (e16 & 1)) * N_NODES

    def drain_scatter(p, b):
        pltpu.make_async_copy(rows_v.at[b], acc_sh.at[sde_v.at[p, b, 1]],
                              ssems[b]).wait()

    def process(p, g, drain):
        pltpu.sync_copy(sde_hbm.at[w, pl.ds(g * K, K)], sde_v.at[p])
        for b in range(K):
            compute_gidx(p, b)
        if drain:
            drain_scatter(p, 0)
        g0 = pltpu.async_copy(tbl_hbm.at[gidx_v.at[0]], rows_v.at[0], ga)
        if drain:
            drain_scatter(p, 1)
        g1 = pltpu.async_copy(tbl_hbm.at[gidx_v.at[1]], rows_v.at[1], gb)
        if drain:
            drain_scatter(p, 2)
        g2 = pltpu.async_copy(tbl_hbm.at[gidx_v.at[2]], rows_v.at[2], gc)
        g0.wait()
        pltpu.async_copy(rows_v.at[0], acc_sh.at[sde_v.at[p, 0, 1]],
                         sa, add=True)
        g1.wait()
        pltpu.async_copy(rows_v.at[1], acc_sh.at[sde_v.at[p, 1, 1]],
                         sb, add=True)
        g2.wait()
        pltpu.async_copy(rows_v.at[2], acc_sh.at[sde_v.at[p, 2, 1]],
                         sc, add=True)

    process(0, 0, drain=False)

    def pair_body(t, carry):
        process(1, 2 * t + 1, drain=True)
        process(0, 2 * t + 2, drain=True)
        return carry

    lax.fori_loop(0, (NG - 2) // 2, pair_body, 0)
    process(1, NG - 1, drain=True)
    for b in range(K):
        drain_scatter(1, b)
    plsc.subcore_barrier()
    pltpu.sync_copy(acc_sh.at[pl.ds(s * ZROWS, ZROWS)],
                    out_hbm.at[c, pl.ds(s * ZROWS, ZROWS)])


assert ACC_ROWS * D + NS * (
    2 * K * 3 * 128 + K * 128 + K * CHUNK * D) < 2097151


def _add_body(p_ref, o_ref):
    o_ref[...] = p_ref[0] + p_ref[1]


def _sum_partials(partials):
    blk = 1000
    return pl.pallas_call(
        _add_body,
        grid=(N_NODES // blk,),
        in_specs=[pl.BlockSpec((2, blk, D), lambda i: (0, i, 0))],
        out_specs=pl.BlockSpec((blk, D), lambda i: (i, 0)),
        out_shape=jax.ShapeDtypeStruct((N_NODES, D), jnp.float32),
    )(partials)


def kernel(graph_embedding, edge_index, e_feat, W):
    tbl = _build_table(graph_embedding, W)

    src = edge_index[0].astype(jnp.int32)
    dst = edge_index[1].astype(jnp.int32)
    e = e_feat.astype(jnp.int32)
    pad = EPAD - N_EDGES
    src_p = jnp.concatenate([src, jnp.zeros((pad,), jnp.int32)])
    dst_p = jnp.concatenate(
        [dst, N_NODES + (jnp.arange(pad, dtype=jnp.int32)
                         % (ACC_ROWS - N_NODES))])
    e_p = jnp.concatenate([e, jnp.ones((pad,), jnp.int32)])
    sde = jnp.stack([src_p.reshape(NW, NCHUNK, CHUNK),
                     dst_p.reshape(NW, NCHUNK, CHUNK),
                     e_p.reshape(NW, NCHUNK, CHUNK)], axis=2)
    sde = jnp.concatenate(
        [sde, jnp.zeros((NW, K, 3, CHUNK), jnp.int32)], axis=1)
    zeros = jnp.zeros((ACC_ROWS, D), jnp.float32)

    partials = _sc_gather_scatter(tbl, sde, zeros)
    return _sum_partials(partials)


# speedup vs baseline: 1.0462x; 1.0462x over previous
import functools

import jax
import jax.numpy as jnp
from jax import lax
from jax.experimental import pallas as pl
from jax.experimental.pallas import tpu as pltpu
from jax.experimental.pallas import tpu_sc as plsc

N_NODES = 10000
N_EDGES = 320000
D = 128

_info = plsc.get_sparse_core_info()
NC = _info.num_cores
NS = _info.num_subcores
L = _info.num_lanes
NW = NC * NS

CHUNK = 112
K = 3
NCHUNK = 90
EPW = NCHUNK * CHUNK
EPAD = EPW * NW
NG = NCHUNK // K

ACC_ROWS = 10112
ZROWS = ACC_ROWS // NS


def _elu_body(x_ref, w_ref, o_ref):
    j = pl.program_id(1)
    y = x_ref[...] * w_ref[...]
    y = jnp.where(y > 0, y, jnp.exp(y) - 1.0)
    o_ref[...] = y * (1.0 + j.astype(jnp.float32))


def _build_table(graph_embedding, W):
    blk = 1000
    grid = (N_NODES // blk, 2)
    return pl.pallas_call(
        _elu_body,
        grid=grid,
        in_specs=[
            pl.BlockSpec((blk, D), lambda i, j: (i, 0)),
            pl.BlockSpec((1, D), lambda i, j: (0, 0)),
        ],
        out_specs=pl.BlockSpec((blk, D), lambda i, j: (j * (N_NODES // blk) + i, 0)),
        out_shape=jax.ShapeDtypeStruct((2 * N_NODES, D), jnp.float32),
    )(graph_embedding, W)


_mesh = plsc.VectorSubcoreMesh(core_axis_name="c", subcore_axis_name="s")


@functools.partial(
    pl.kernel,
    mesh=_mesh,
    out_type=jax.ShapeDtypeStruct((NC, ACC_ROWS, D), jnp.float32),
    scratch_types=[
        pltpu.VMEM((2, K, 3, CHUNK), jnp.int32),
        pltpu.VMEM((K, CHUNK), jnp.int32),
        pltpu.VMEM((K, CHUNK, D), jnp.float32),
        pltpu.VMEM_SHARED((ACC_ROWS, D), jnp.float32),
        pltpu.SemaphoreType.DMA,
        pltpu.SemaphoreType.DMA,
        pltpu.SemaphoreType.DMA,
        pltpu.SemaphoreType.DMA,
        pltpu.SemaphoreType.DMA,
        pltpu.SemaphoreType.DMA,
    ],
)
def _sc_gather_scatter(tbl_hbm, sde_hbm, zeros_hbm, out_hbm,
                       sde_v, gidx_v, rows_v, acc_sh,
                       ga, gb, gc, sa, sb, sc):
    c = lax.axis_index("c")
    s = lax.axis_index("s")
    w = c * NS + s
    gsems = (ga, gb, gc)
    ssems = (sa, sb, sc)

    pltpu.sync_copy(zeros_hbm.at[pl.ds(s * ZROWS, ZROWS)],
                    acc_sh.at[pl.ds(s * ZROWS, ZROWS)])
    plsc.subcore_barrier()

    def compute_gidx(p, b):
        for k in range(CHUNK // L):
            sl = pl.ds(k * L, L)
            s16 = sde_v[p, b, 0, sl]
            e16 = sde_v[p, b, 2, sl]
            gidx_v[b, sl] = s16 + (1 - (e16 & 1)) * N_NODES

    def drain_scatter(p, b):
        pltpu.make_async_copy(rows_v.at[b], acc_sh.at[sde_v.at[p, b, 1]],
                              ssems[b]).wait()

    def process(p, g, drain):
        pltpu.sync_copy(sde_hbm.at[w, pl.ds(g * K, K)], sde_v.at[p])
        for b in range(K):
            compute_gidx(p, b)
        if drain:
            drain_scatter(p, 0)
        g0 = pltpu.async_copy(tbl_hbm.at[gidx_v.at[0]], rows_v.at[0], ga)
        if drain:
            drain_scatter(p, 1)
        g1 = pltpu.async_copy(tbl_hbm.at[gidx_v.at[1]], rows_v.at[1], gb)
        if drain:
            drain_scatter(p, 2)
        g2 = pltpu.async_copy(tbl_hbm.at[gidx_v.at[2]], rows_v.at[2], gc)
        g0.wait()
        pltpu.async_copy(rows_v.at[0], acc_sh.at[sde_v.at[p, 0, 1]],
                         sa, add=True)
        g1.wait()
        pltpu.async_copy(rows_v.at[1], acc_sh.at[sde_v.at[p, 1, 1]],
                         sb, add=True)
        g2.wait()
        pltpu.async_copy(rows_v.at[2], acc_sh.at[sde_v.at[p, 2, 1]],
                         sc, add=True)

    process(0, 0, drain=False)

    def pair_body(t, carry):
        process(1, 2 * t + 1, drain=True)
        process(0, 2 * t + 2, drain=True)
        return carry

    lax.fori_loop(0, (NG - 2) // 2, pair_body, 0)
    process(1, NG - 1, drain=True)
    for b in range(K):
        drain_scatter(1, b)
    plsc.subcore_barrier()
    pltpu.sync_copy(acc_sh.at[pl.ds(s * ZROWS, ZROWS)],
                    out_hbm.at[c, pl.ds(s * ZROWS, ZROWS)])


assert ACC_ROWS * D + NS * (
    2 * K * 3 * 128 + K * 128 + K * CHUNK * D) < 2097151


def _add_body(p_ref, o_ref):
    o_ref[...] = p_ref[0] + p_ref[1]


def _sum_partials(partials):
    blk = 1000
    return pl.pallas_call(
        _add_body,
        grid=(N_NODES // blk,),
        in_specs=[pl.BlockSpec((2, blk, D), lambda i: (0, i, 0))],
        out_specs=pl.BlockSpec((blk, D), lambda i: (i, 0)),
        out_shape=jax.ShapeDtypeStruct((N_NODES, D), jnp.float32),
    )(partials)


def kernel(graph_embedding, edge_index, e_feat, W):
    tbl = _build_table(graph_embedding, W)

    src = edge_index[0].astype(jnp.int32)
    dst = edge_index[1].astype(jnp.int32)
    e = e_feat.astype(jnp.int32)
    pad = EPAD - N_EDGES
    src_p = jnp.concatenate([src, jnp.zeros((pad,), jnp.int32)])
    dst_p = jnp.concatenate(
        [dst, N_NODES + (jnp.arange(pad, dtype=jnp.int32)
                         % (ACC_ROWS - N_NODES))])
    e_p = jnp.concatenate([e, jnp.ones((pad,), jnp.int32)])
    sde = jnp.stack([src_p.reshape(NW, NCHUNK, CHUNK),
                     dst_p.reshape(NW, NCHUNK, CHUNK),
                     e_p.reshape(NW, NCHUNK, CHUNK)], axis=2)
    zeros = jnp.zeros((ACC_ROWS, D), jnp.float32)

    partials = _sc_gather_scatter(tbl, sde, zeros)
    return _sum_partials(partials)
